# Initial kernel scaffold; baseline (speedup 1.0000x reference)
import jax
import jax.numpy as jnp
from jax.experimental import pallas as pl


def kernel(x, table):
    raise NotImplementedError("write your pallas kernel here")


# speedup vs baseline: 3.2743x; 3.2743x over previous
import functools

import jax
import jax.numpy as jnp
from jax import lax
from jax.experimental import pallas as pl
from jax.experimental.pallas import tpu as pltpu
from jax.experimental.pallas import tpu_sc as plsc


def _make_gather(n_total, vocab, dim, n_workers, num_cores, chunk):
    n_per_w = n_total // n_workers
    n_chunks = n_per_w // chunk
    mesh = plsc.VectorSubcoreMesh(core_axis_name="c", subcore_axis_name="s")

    @functools.partial(
        pl.kernel,
        mesh=mesh,
        out_type=jax.ShapeDtypeStruct((n_total, dim), jnp.float32),
        scratch_types=[
            pltpu.VMEM((chunk,), jnp.int32),
            pltpu.VMEM((chunk, dim), jnp.float32),
            pltpu.SemaphoreType.DMA,
        ],
    )
    def emb(table_hbm, idx_hbm, out_hbm, idx_v, rows_v, sem):
        wid = lax.axis_index("s") * num_cores + lax.axis_index("c")
        base = wid * n_per_w

        def body(i, carry):
            off = pl.multiple_of(base + i * chunk, 8)
            pltpu.sync_copy(idx_hbm.at[pl.ds(off, chunk)], idx_v)
            pltpu.async_copy(table_hbm.at[idx_v], rows_v, sem).wait()
            pltpu.sync_copy(rows_v, out_hbm.at[pl.ds(off, chunk)])
            return carry

        lax.fori_loop(0, n_chunks, body, 0)

    return emb


def kernel(x, table):
    batch, hist = x.shape
    vocab, dim = table.shape
    n_total = batch * hist
    idx = x.reshape(n_total).astype(jnp.int32)

    info = plsc.get_sparse_core_info()
    n_workers = info.num_cores * info.num_subcores
    chunk = 640

    emb = _make_gather(n_total, vocab, dim, n_workers, info.num_cores, chunk)
    out = emb(table, idx)
    return out.reshape(batch, hist, dim)


# speedup vs baseline: 3.3436x; 1.0212x over previous
import functools

import jax
import jax.numpy as jnp
from jax import lax
from jax.experimental import pallas as pl
from jax.experimental.pallas import tpu as pltpu
from jax.experimental.pallas import tpu_sc as plsc


def _make_gather(n_total, vocab, dim, n_workers, num_cores, chunk):
    n_per_w = n_total // n_workers
    n_chunks = n_per_w // chunk
    mesh = plsc.VectorSubcoreMesh(core_axis_name="c", subcore_axis_name="s")

    @functools.partial(
        pl.kernel,
        mesh=mesh,
        out_type=jax.ShapeDtypeStruct((n_total, dim), jnp.float32),
        scratch_types=[
            pltpu.VMEM((n_per_w,), jnp.int32),
            pltpu.VMEM((2, chunk, dim), jnp.float32),
            pltpu.SemaphoreType.DMA,
            pltpu.SemaphoreType.DMA,
        ],
    )
    def emb(table_hbm, idx_hbm, out_hbm, idx_v, rows_v, gsem, wsem):
        wid = lax.axis_index("s") * num_cores + lax.axis_index("c")
        base = wid * n_per_w
        pltpu.sync_copy(idx_hbm.at[pl.ds(base, n_per_w)], idx_v)

        def start_gather(i):
            return pltpu.async_copy(
                table_hbm.at[idx_v.at[pl.ds(i * chunk, chunk)]],
                rows_v.at[i % 2],
                gsem,
            )

        gathers = [None] * n_chunks
        writes = [None] * n_chunks
        gathers[0] = start_gather(0)
        for i in range(n_chunks):
            if i + 1 < n_chunks:
                if i >= 1:
                    writes[i - 1].wait()
                gathers[i + 1] = start_gather(i + 1)
            gathers[i].wait()
            writes[i] = pltpu.async_copy(
                rows_v.at[i % 2],
                out_hbm.at[pl.ds(base + i * chunk, chunk)],
                wsem,
            )
        if n_chunks >= 2:
            writes[n_chunks - 2].wait()
        writes[n_chunks - 1].wait()

    return emb


def kernel(x, table):
    batch, hist = x.shape
    vocab, dim = table.shape
    n_total = batch * hist
    idx = x.reshape(n_total).astype(jnp.int32)

    info = plsc.get_sparse_core_info()
    n_workers = info.num_cores * info.num_subcores
    chunk = 400

    emb = _make_gather(n_total, vocab, dim, n_workers, info.num_cores, chunk)
    out = emb(table, idx)
    return out.reshape(batch, hist, dim)


# speedup vs baseline: 5.9298x; 1.7735x over previous
import functools

import jax
import jax.numpy as jnp
from jax import lax
from jax.experimental import pallas as pl
from jax.experimental.pallas import tpu as pltpu
from jax.experimental.pallas import tpu_sc as plsc


def _make_gather(batch, hist, vocab, dim, n_workers, num_cores, rows_chunk):
    n_total = batch * hist
    n_per_w = n_total // n_workers
    b_per_w = batch // n_workers
    chunk = rows_chunk * hist
    n_chunks = b_per_w // rows_chunk
    mesh = plsc.VectorSubcoreMesh(core_axis_name="c", subcore_axis_name="s")

    @functools.partial(
        pl.kernel,
        mesh=mesh,
        out_type=jax.ShapeDtypeStruct((batch, hist, dim), jnp.float32),
        scratch_types=[
            pltpu.VMEM((n_per_w,), jnp.int32),
            pltpu.VMEM((2, chunk, dim), jnp.float32),
            pltpu.SemaphoreType.DMA,
            pltpu.SemaphoreType.DMA,
        ],
    )
    def emb(table_hbm, idx_hbm, out_hbm, idx_v, rows_v, gsem, wsem):
        wid = lax.axis_index("s") * num_cores + lax.axis_index("c")
        base = wid * n_per_w
        brow = wid * b_per_w
        pltpu.sync_copy(idx_hbm.at[pl.ds(base, n_per_w)], idx_v)

        def start_gather(i):
            return pltpu.async_copy(
                table_hbm.at[idx_v.at[pl.ds(i * chunk, chunk)]],
                rows_v.at[i % 2],
                gsem,
            )

        gathers = [None] * n_chunks
        writes = [None] * n_chunks
        gathers[0] = start_gather(0)
        for i in range(n_chunks):
            if i + 1 < n_chunks:
                if i >= 1:
                    writes[i - 1].wait()
                gathers[i + 1] = start_gather(i + 1)
            gathers[i].wait()
            writes[i] = pltpu.async_copy(
                rows_v.at[i % 2].reshape(rows_chunk, hist, dim),
                out_hbm.at[pl.ds(brow + i * rows_chunk, rows_chunk)],
                wsem,
            )
        if n_chunks >= 2:
            writes[n_chunks - 2].wait()
        writes[n_chunks - 1].wait()

    return emb


def kernel(x, table):
    batch, hist = x.shape
    vocab, dim = table.shape
    idx = x.reshape(batch * hist).astype(jnp.int32)

    info = plsc.get_sparse_core_info()
    n_workers = info.num_cores * info.num_subcores
    rows_chunk = 8

    emb = _make_gather(
        batch, hist, vocab, dim, n_workers, info.num_cores, rows_chunk
    )
    return emb(table, idx)
